# baseline (speedup 1.0000x reference)
import functools

import jax
import jax.numpy as jnp
from jax import lax
from jax.experimental import pallas as pl
from jax.experimental.pallas import tpu as pltpu
from jax.experimental.pallas import tpu_sc as plsc

NC = 2
NS = 16
CHUNK = 128
NBUF = 6
ZROWS = 32


def _segment_sum_sc(n_nodes, feat, e_pad):
    hf = feat // 2
    trows = ((n_nodes + NS - 1) // NS + ZROWS - 1) // ZROWS * ZROWS
    n_acc = NS * trows
    nchunks = e_pad // CHUNK
    tile_chunks = nchunks // NS
    ntrip = tile_chunks // NBUF
    last_rows = n_nodes - (NS - 1) * trows

    mesh = plsc.VectorSubcoreMesh(core_axis_name="c", subcore_axis_name="s",
                                  num_cores=NC, num_subcores=NS)

    def body(ft3, src2d, dst2d, out, acc, src, draw, rows, zbuf,
             gsem, ssem, isem):
        c = lax.axis_index("c")
        s = lax.axis_index("s")
        cbase = s * tile_chunks
        ftv = ft3.at[c]

        def zrow(i, _):
            for k in range(hf // 16):
                zbuf[i, pl.ds(k * 16, 16)] = jnp.zeros((16,), jnp.float32)
            return 0
        lax.fori_loop(0, ZROWS, zrow, 0)

        abase = s * trows
        def zacc(i, _):
            pltpu.sync_copy(zbuf, acc.at[pl.ds(abase + i * ZROWS, ZROWS)])
            return 0
        lax.fori_loop(0, trows // ZROWS, zacc, 0)
        plsc.subcore_barrier()

        def fire_idx(chunk, b):
            pltpu.async_copy(src2d.at[pl.ds(chunk, 1)], src.at[b], isem[b])
            pltpu.async_copy(dst2d.at[pl.ds(chunk, 1)], draw.at[b], isem[b])

        def wait_idx(chunk, b):
            pltpu.make_async_copy(src2d.at[pl.ds(chunk, 1)], src.at[b],
                                  isem[b]).wait()
            pltpu.make_async_copy(dst2d.at[pl.ds(chunk, 1)], draw.at[b],
                                  isem[b]).wait()

        def fire_gather(b):
            pltpu.async_copy(ftv.at[src.at[b, 0]], rows.at[b], gsem[b])

        def wait_gather(b):
            pltpu.make_async_copy(ftv.at[src.at[b, 0]], rows.at[b],
                                  gsem[b]).wait()

        def fire_scatter(b):
            pltpu.async_copy(rows.at[b], acc.at[draw.at[b, 0]], ssem[b],
                             add=True)

        def wait_scatter(b):
            pltpu.make_async_copy(rows.at[b], acc.at[draw.at[b, 0]],
                                  ssem[b]).wait()

        def _maybe(cond, fn):
            if cond is True:
                fn()
            else:
                pl.when(cond)(fn)

        pltpu.sync_copy(src2d.at[pl.ds(cbase, 1)], src.at[0])
        pltpu.sync_copy(dst2d.at[pl.ds(cbase, 1)], draw.at[0])
        fire_gather(0)
        fire_idx(cbase + 1, 1)

        def trip(t, _):
            c0 = cbase + NBUF * t
            for k in range(NBUF):
                cur = k
                nxt = (k + 1) % NBUF
                nx2 = (k + 2) % NBUF
                ck = c0 + k
                has1 = True if k < NBUF - 1 else t < ntrip - 1
                has2 = True if k < NBUF - 2 else t < ntrip - 1
                not_first = True if k == NBUF - 1 else t > 0
                _maybe(not_first, lambda nxt=nxt: wait_scatter(nxt))
                def _g(ck=ck, nxt=nxt):
                    wait_idx(ck + 1, nxt)
                    fire_gather(nxt)
                _maybe(has1, _g)
                _maybe(has2, lambda ck=ck, nx2=nx2: fire_idx(ck + 2, nx2))
                wait_gather(cur)
                fire_scatter(cur)
            return 0
        lax.fori_loop(0, ntrip, trip, 0)
        for j in range(1, NBUF):
            wait_scatter((tile_chunks - j) % NBUF)

        plsc.subcore_barrier()

        obase = s * trows
        @pl.when(s < NS - 1)
        def _():
            pltpu.sync_copy(acc.at[pl.ds(abase, trows)],
                            out.at[pl.ds(obase, trows), pl.ds(c * hf, hf)])
        @pl.when(s == NS - 1)
        def _():
            pltpu.sync_copy(acc.at[pl.ds(abase, last_rows)],
                            out.at[pl.ds(obase, last_rows), pl.ds(c * hf, hf)])

    return pl.kernel(
        body,
        out_type=jax.ShapeDtypeStruct((n_nodes, feat), jnp.float32),
        mesh=mesh,
        compiler_params=pltpu.CompilerParams(use_tc_tiling_on_sc=False),
        scratch_types=[
            pltpu.VMEM_SHARED((n_acc, hf), jnp.float32),
            pltpu.VMEM((NBUF, 1, CHUNK), jnp.int32),
            pltpu.VMEM((NBUF, 1, CHUNK), jnp.int32),
            pltpu.VMEM((NBUF, CHUNK, hf), jnp.float32),
            pltpu.VMEM((ZROWS, hf), jnp.float32),
            [pltpu.SemaphoreType.DMA] * NBUF,
            [pltpu.SemaphoreType.DMA] * NBUF,
            [pltpu.SemaphoreType.DMA] * NBUF,
        ],
    ), n_acc, n_nodes


def _dense1_body(x_ref, a_ref, wa_ref, wb_ref, o_ref):
    x = x_ref[...]
    a = a_ref[...]
    dn = (((1,), (1,)), ((), ()))
    z = lax.dot_general(x + a, wa_ref[...], dn,
                        preferred_element_type=jnp.float32)
    z = z + lax.dot_general(x * a, wb_ref[...], dn,
                            preferred_element_type=jnp.float32)
    o_ref[...] = jnp.maximum(z, 0.0)


def _dense2_body(x_ref, a_ref, wa_ref, wb_ref, o_ref):
    x = x_ref[...]
    a = a_ref[...]
    dn = (((1,), (1,)), ((), ()))
    z = lax.dot_general(x + a, wa_ref[...], dn,
                        preferred_element_type=jnp.float32)
    z = z + lax.dot_general(x * a, wb_ref[...], dn,
                            preferred_element_type=jnp.float32)
    h = jnp.maximum(z, 0.0)
    nrm = jnp.sqrt(jnp.sum(h * h, axis=1, keepdims=True))
    o_ref[...] = h / jnp.maximum(nrm, 1e-12)


def _dense(body, x, agg, w, rows_blk):
    n, f = x.shape
    grid = n // rows_blk
    wa = w[:, :f]
    wb = w[:, f:]
    return pl.pallas_call(
        body,
        grid=(grid,),
        in_specs=[
            pl.BlockSpec((rows_blk, f), lambda i: (i, 0)),
            pl.BlockSpec((rows_blk, f), lambda i: (i, 0)),
            pl.BlockSpec((f, f), lambda i: (0, 0)),
            pl.BlockSpec((f, f), lambda i: (0, 0)),
        ],
        out_specs=pl.BlockSpec((rows_blk, f), lambda i: (i, 0)),
        out_shape=jax.ShapeDtypeStruct((n, f), jnp.float32),
    )(x, agg, wa, wb)


@functools.partial(jax.jit, static_argnums=())
def kernel(node_l2, node_l1, ft_lv0, edge_index, W1, W2):
    n, f = ft_lv0.shape
    e = edge_index.shape[1]

    unit = NS * CHUNK * NBUF
    e_pad = (e + unit - 1) // unit * unit
    src = edge_index[0].astype(jnp.int32)
    dst = edge_index[1].astype(jnp.int32)
    src_p = jnp.concatenate([src, jnp.zeros((e_pad - e,), jnp.int32)])
    dst_p = jnp.concatenate([dst, jnp.full((e_pad - e,), n, jnp.int32)])
    src2d = src_p.reshape(e_pad // CHUNK, CHUNK)
    dst2d = dst_p.reshape(e_pad // CHUNK, CHUNK)

    seg, _, _ = _segment_sum_sc(n, f, e_pad)
    hf = f // 2

    def segsum(x):
        x3 = jnp.stack([x[:, :hf], x[:, hf:]])
        return seg(x3, src2d, dst2d)

    agg0 = segsum(ft_lv0)
    ft1 = _dense(_dense1_body, ft_lv0, agg0, W1, 2000)
    agg1 = segsum(ft1)
    out = _dense(_dense2_body, ft1, agg1, W2, 2000)
    return out


# speedup vs baseline: 1.2374x; 1.2374x over previous
import functools

import jax
import jax.numpy as jnp
from jax import lax
from jax.experimental import pallas as pl
from jax.experimental.pallas import tpu as pltpu
from jax.experimental.pallas import tpu_sc as plsc

NC = 2
NS = 16
CHUNK = 128
NBUF = 3
ZROWS = 32


def _segment_sum_sc(n_nodes, feat, e_pad, dtype=jnp.float32):
    hf = feat // 2
    trows = ((n_nodes + NS - 1) // NS + ZROWS - 1) // ZROWS * ZROWS
    n_acc = NS * trows
    nchunks = e_pad // CHUNK
    tile_chunks = nchunks // NS
    ntrip = tile_chunks // NBUF
    last_rows = n_nodes - (NS - 1) * trows

    mesh = plsc.VectorSubcoreMesh(core_axis_name="c", subcore_axis_name="s",
                                  num_cores=NC, num_subcores=NS)

    def body(ft3, src2d, dst2d, out, acc, src, draw, rows, zbuf,
             gsem, ssem, isem):
        c = lax.axis_index("c")
        s = lax.axis_index("s")
        cbase = s * tile_chunks
        ftv = ft3.at[c]

        lanes = 32 if dtype == jnp.bfloat16 else 16
        def zrow(i, _):
            for k in range(hf // lanes):
                zbuf[i, pl.ds(k * lanes, lanes)] = jnp.zeros((lanes,), dtype)
            return 0
        lax.fori_loop(0, ZROWS, zrow, 0)

        abase = s * trows
        def zacc(i, _):
            pltpu.sync_copy(zbuf, acc.at[pl.ds(abase + i * ZROWS, ZROWS)])
            return 0
        lax.fori_loop(0, trows // ZROWS, zacc, 0)
        plsc.subcore_barrier()

        def fire_idx(chunk, b):
            pltpu.async_copy(src2d.at[pl.ds(chunk, 1)], src.at[b], isem[b])
            pltpu.async_copy(dst2d.at[pl.ds(chunk, 1)], draw.at[b], isem[b])

        def wait_idx(chunk, b):
            pltpu.make_async_copy(src2d.at[pl.ds(chunk, 1)], src.at[b],
                                  isem[b]).wait()
            pltpu.make_async_copy(dst2d.at[pl.ds(chunk, 1)], draw.at[b],
                                  isem[b]).wait()

        def fire_gather(b):
            pltpu.async_copy(ftv.at[src.at[b, 0]], rows.at[b], gsem[b])

        def wait_gather(b):
            pltpu.make_async_copy(ftv.at[src.at[b, 0]], rows.at[b],
                                  gsem[b]).wait()

        def fire_scatter(b):
            pltpu.async_copy(rows.at[b], acc.at[draw.at[b, 0]], ssem[b],
                             add=True)

        def wait_scatter(b):
            pltpu.make_async_copy(rows.at[b], acc.at[draw.at[b, 0]],
                                  ssem[b]).wait()

        def _maybe(cond, fn):
            if cond is True:
                fn()
            else:
                pl.when(cond)(fn)

        pltpu.sync_copy(src2d.at[pl.ds(cbase, 1)], src.at[0])
        pltpu.sync_copy(dst2d.at[pl.ds(cbase, 1)], draw.at[0])
        fire_gather(0)
        fire_idx(cbase + 1, 1)

        def trip(t, _):
            c0 = cbase + NBUF * t
            for k in range(NBUF):
                cur = k
                nxt = (k + 1) % NBUF
                nx2 = (k + 2) % NBUF
                ck = c0 + k
                has1 = True if k < NBUF - 1 else t < ntrip - 1
                has2 = True if k < NBUF - 2 else t < ntrip - 1
                not_first = True if k == NBUF - 1 else t > 0
                _maybe(not_first, lambda nxt=nxt: wait_scatter(nxt))
                def _g(ck=ck, nxt=nxt):
                    wait_idx(ck + 1, nxt)
                    fire_gather(nxt)
                _maybe(has1, _g)
                _maybe(has2, lambda ck=ck, nx2=nx2: fire_idx(ck + 2, nx2))
                wait_gather(cur)
                fire_scatter(cur)
            return 0
        lax.fori_loop(0, ntrip, trip, 0)
        for j in range(1, NBUF):
            wait_scatter((tile_chunks - j) % NBUF)

        plsc.subcore_barrier()

        obase = s * trows
        @pl.when(s < NS - 1)
        def _():
            pltpu.sync_copy(acc.at[pl.ds(abase, trows)],
                            out.at[pl.ds(obase, trows), pl.ds(c * hf, hf)])
        @pl.when(s == NS - 1)
        def _():
            pltpu.sync_copy(acc.at[pl.ds(abase, last_rows)],
                            out.at[pl.ds(obase, last_rows), pl.ds(c * hf, hf)])

    return pl.kernel(
        body,
        out_type=jax.ShapeDtypeStruct((n_nodes, feat), dtype),
        mesh=mesh,
        compiler_params=pltpu.CompilerParams(use_tc_tiling_on_sc=False),
        scratch_types=[
            pltpu.VMEM_SHARED((n_acc, hf), dtype),
            pltpu.VMEM((NBUF, 1, CHUNK), jnp.int32),
            pltpu.VMEM((NBUF, 1, CHUNK), jnp.int32),
            pltpu.VMEM((NBUF, CHUNK, hf), dtype),
            pltpu.VMEM((ZROWS, hf), dtype),
            [pltpu.SemaphoreType.DMA] * NBUF,
            [pltpu.SemaphoreType.DMA] * NBUF,
            [pltpu.SemaphoreType.DMA] * NBUF,
        ],
    ), n_acc, n_nodes


def _dense1_body(x_ref, a_ref, wa_ref, wb_ref, o_ref):
    x = x_ref[...]
    a = a_ref[...].astype(jnp.float32)
    dn = (((1,), (1,)), ((), ()))
    z = lax.dot_general(x + a, wa_ref[...], dn,
                        preferred_element_type=jnp.float32)
    z = z + lax.dot_general(x * a, wb_ref[...], dn,
                            preferred_element_type=jnp.float32)
    o_ref[...] = jnp.maximum(z, 0.0)


def _dense2_body(x_ref, a_ref, wa_ref, wb_ref, o_ref):
    x = x_ref[...]
    a = a_ref[...].astype(jnp.float32)
    dn = (((1,), (1,)), ((), ()))
    z = lax.dot_general(x + a, wa_ref[...], dn,
                        preferred_element_type=jnp.float32)
    z = z + lax.dot_general(x * a, wb_ref[...], dn,
                            preferred_element_type=jnp.float32)
    h = jnp.maximum(z, 0.0)
    nrm = jnp.sqrt(jnp.sum(h * h, axis=1, keepdims=True))
    o_ref[...] = h / jnp.maximum(nrm, 1e-12)


def _dense(body, x, agg, w, rows_blk):
    n, f = x.shape
    grid = n // rows_blk
    wa = w[:, :f]
    wb = w[:, f:]
    return pl.pallas_call(
        body,
        grid=(grid,),
        in_specs=[
            pl.BlockSpec((rows_blk, f), lambda i: (i, 0)),
            pl.BlockSpec((rows_blk, f), lambda i: (i, 0)),
            pl.BlockSpec((f, f), lambda i: (0, 0)),
            pl.BlockSpec((f, f), lambda i: (0, 0)),
        ],
        out_specs=pl.BlockSpec((rows_blk, f), lambda i: (i, 0)),
        out_shape=jax.ShapeDtypeStruct((n, f), jnp.float32),
    )(x, agg, wa, wb)


@functools.partial(jax.jit, static_argnums=())
def kernel(node_l2, node_l1, ft_lv0, edge_index, W1, W2):
    n, f = ft_lv0.shape
    e = edge_index.shape[1]

    unit = NS * CHUNK * NBUF
    e_pad = (e + unit - 1) // unit * unit
    src = edge_index[0].astype(jnp.int32)
    dst = edge_index[1].astype(jnp.int32)
    src_p = jnp.concatenate([src, jnp.zeros((e_pad - e,), jnp.int32)])
    dst_p = jnp.concatenate([dst, jnp.full((e_pad - e,), n, jnp.int32)])
    src2d = src_p.reshape(e_pad // CHUNK, CHUNK)
    dst2d = dst_p.reshape(e_pad // CHUNK, CHUNK)

    seg, _, _ = _segment_sum_sc(n, f, e_pad, jnp.bfloat16)
    hf = f // 2

    def segsum(x):
        xb = x.astype(jnp.bfloat16)
        x3 = jnp.stack([xb[:, :hf], xb[:, hf:]])
        return seg(x3, src2d, dst2d)

    agg0 = segsum(ft_lv0)
    ft1 = _dense(_dense1_body, ft_lv0, agg0, W1, 2000)
    agg1 = segsum(ft1)
    out = _dense(_dense2_body, ft1, agg1, W2, 2000)
    return out


# speedup vs baseline: 1.4825x; 1.1981x over previous
import functools

import jax
import jax.numpy as jnp
from jax import lax
from jax.experimental import pallas as pl
from jax.experimental.pallas import tpu as pltpu
from jax.experimental.pallas import tpu_sc as plsc

NC = 2
NS = 16
CHUNK = 128
NBUF = 3
GK = 4
ZROWS = 32


def _segment_sum_sc(n_nodes, feat, e_pad, dtype=jnp.float32):
    hf = feat // 2
    trows = ((n_nodes + NS - 1) // NS + ZROWS - 1) // ZROWS * ZROWS
    n_acc = NS * trows
    nchunks = e_pad // (GK * CHUNK)
    tile_chunks = nchunks // NS
    ntrip = tile_chunks // NBUF
    last_rows = n_nodes - (NS - 1) * trows

    mesh = plsc.VectorSubcoreMesh(core_axis_name="c", subcore_axis_name="s",
                                  num_cores=NC, num_subcores=NS)

    def body(ft3, src2d, dst2d, out, acc, src, draw, rows, zbuf,
             gsem, ssem, isem):
        c = lax.axis_index("c")
        s = lax.axis_index("s")
        cbase = s * tile_chunks
        ftv = ft3.at[c]

        lanes = 32 if dtype == jnp.bfloat16 else 16
        def zrow(i, _):
            for k in range(hf // lanes):
                zbuf[i, pl.ds(k * lanes, lanes)] = jnp.zeros((lanes,), dtype)
            return 0
        lax.fori_loop(0, ZROWS, zrow, 0)

        abase = s * trows
        def zacc(i, _):
            pltpu.sync_copy(zbuf, acc.at[pl.ds(abase + i * ZROWS, ZROWS)])
            return 0
        lax.fori_loop(0, trows // ZROWS, zacc, 0)
        plsc.subcore_barrier()

        def fire_idx(chunk, b):
            pltpu.async_copy(src2d.at[pl.ds(chunk, 1)], src.at[b], isem[b])
            pltpu.async_copy(dst2d.at[pl.ds(chunk, 1)], draw.at[b], isem[b])

        def wait_idx(chunk, b):
            pltpu.make_async_copy(src2d.at[pl.ds(chunk, 1)], src.at[b],
                                  isem[b]).wait()
            pltpu.make_async_copy(dst2d.at[pl.ds(chunk, 1)], draw.at[b],
                                  isem[b]).wait()

        def fire_gather(b):
            pltpu.async_copy(ftv.at[src.at[b, 0]], rows.at[b], gsem[b])

        def wait_gather(b):
            pltpu.make_async_copy(ftv.at[src.at[b, 0]], rows.at[b],
                                  gsem[b]).wait()

        def fire_scatter(b):
            for j in range(GK):
                pltpu.async_copy(rows.at[b, pl.ds(j * CHUNK, CHUNK)],
                                 acc.at[draw.at[b, 0, j]], ssem[b], add=True)

        def wait_scatter(b):
            for j in range(GK):
                pltpu.make_async_copy(rows.at[b, pl.ds(j * CHUNK, CHUNK)],
                                      acc.at[draw.at[b, 0, j]],
                                      ssem[b]).wait()

        def _maybe(cond, fn):
            if cond is True:
                fn()
            else:
                pl.when(cond)(fn)

        pltpu.sync_copy(src2d.at[pl.ds(cbase, 1)], src.at[0])
        pltpu.sync_copy(dst2d.at[pl.ds(cbase, 1)], draw.at[0])
        fire_gather(0)
        fire_idx(cbase + 1, 1)

        def trip(t, _):
            c0 = cbase + NBUF * t
            for k in range(NBUF):
                cur = k
                nxt = (k + 1) % NBUF
                nx2 = (k + 2) % NBUF
                ck = c0 + k
                has1 = True if k < NBUF - 1 else t < ntrip - 1
                has2 = True if k < NBUF - 2 else t < ntrip - 1
                not_first = True if k == NBUF - 1 else t > 0
                _maybe(not_first, lambda nxt=nxt: wait_scatter(nxt))
                def _g(ck=ck, nxt=nxt):
                    wait_idx(ck + 1, nxt)
                    fire_gather(nxt)
                _maybe(has1, _g)
                _maybe(has2, lambda ck=ck, nx2=nx2: fire_idx(ck + 2, nx2))
                wait_gather(cur)
                fire_scatter(cur)
            return 0
        lax.fori_loop(0, ntrip, trip, 0)
        for j in range(1, NBUF):
            wait_scatter((tile_chunks - j) % NBUF)

        plsc.subcore_barrier()

        obase = s * trows
        @pl.when(s < NS - 1)
        def _():
            pltpu.sync_copy(acc.at[pl.ds(abase, trows)],
                            out.at[pl.ds(obase, trows), pl.ds(c * hf, hf)])
        @pl.when(s == NS - 1)
        def _():
            pltpu.sync_copy(acc.at[pl.ds(abase, last_rows)],
                            out.at[pl.ds(obase, last_rows), pl.ds(c * hf, hf)])

    return pl.kernel(
        body,
        out_type=jax.ShapeDtypeStruct((n_nodes, feat), dtype),
        mesh=mesh,
        compiler_params=pltpu.CompilerParams(use_tc_tiling_on_sc=False),
        scratch_types=[
            pltpu.VMEM_SHARED((n_acc, hf), dtype),
            pltpu.VMEM((NBUF, 1, GK * CHUNK), jnp.int32),
            pltpu.VMEM((NBUF, 1, GK, CHUNK), jnp.int32),
            pltpu.VMEM((NBUF, GK * CHUNK, hf), dtype),
            pltpu.VMEM((ZROWS, hf), dtype),
            [pltpu.SemaphoreType.DMA] * NBUF,
            [pltpu.SemaphoreType.DMA] * NBUF,
            [pltpu.SemaphoreType.DMA] * NBUF,
        ],
    ), n_acc, n_nodes


def _dense1_body(x_ref, a_ref, wa_ref, wb_ref, o_ref):
    x = x_ref[...]
    a = a_ref[...].astype(jnp.float32)
    dn = (((1,), (1,)), ((), ()))
    z = lax.dot_general(x + a, wa_ref[...], dn,
                        preferred_element_type=jnp.float32)
    z = z + lax.dot_general(x * a, wb_ref[...], dn,
                            preferred_element_type=jnp.float32)
    o_ref[...] = jnp.maximum(z, 0.0)


def _dense2_body(x_ref, a_ref, wa_ref, wb_ref, o_ref):
    x = x_ref[...]
    a = a_ref[...].astype(jnp.float32)
    dn = (((1,), (1,)), ((), ()))
    z = lax.dot_general(x + a, wa_ref[...], dn,
                        preferred_element_type=jnp.float32)
    z = z + lax.dot_general(x * a, wb_ref[...], dn,
                            preferred_element_type=jnp.float32)
    h = jnp.maximum(z, 0.0)
    nrm = jnp.sqrt(jnp.sum(h * h, axis=1, keepdims=True))
    o_ref[...] = h / jnp.maximum(nrm, 1e-12)


def _dense(body, x, agg, w, rows_blk):
    n, f = x.shape
    grid = n // rows_blk
    wa = w[:, :f]
    wb = w[:, f:]
    return pl.pallas_call(
        body,
        grid=(grid,),
        in_specs=[
            pl.BlockSpec((rows_blk, f), lambda i: (i, 0)),
            pl.BlockSpec((rows_blk, f), lambda i: (i, 0)),
            pl.BlockSpec((f, f), lambda i: (0, 0)),
            pl.BlockSpec((f, f), lambda i: (0, 0)),
        ],
        out_specs=pl.BlockSpec((rows_blk, f), lambda i: (i, 0)),
        out_shape=jax.ShapeDtypeStruct((n, f), jnp.float32),
    )(x, agg, wa, wb)


@functools.partial(jax.jit, static_argnums=())
def kernel(node_l2, node_l1, ft_lv0, edge_index, W1, W2):
    n, f = ft_lv0.shape
    e = edge_index.shape[1]

    unit = NS * CHUNK * GK * NBUF
    e_pad = (e + unit - 1) // unit * unit
    src = edge_index[0].astype(jnp.int32)
    dst = edge_index[1].astype(jnp.int32)
    src_p = jnp.concatenate([src, jnp.zeros((e_pad - e,), jnp.int32)])
    dst_p = jnp.concatenate([dst, jnp.full((e_pad - e,), n, jnp.int32)])
    src2d = src_p.reshape(e_pad // (GK * CHUNK), GK * CHUNK)
    dst2d = dst_p.reshape(e_pad // (GK * CHUNK), GK, CHUNK)

    seg, _, _ = _segment_sum_sc(n, f, e_pad, jnp.bfloat16)
    hf = f // 2

    def segsum(x):
        xb = x.astype(jnp.bfloat16)
        x3 = jnp.stack([xb[:, :hf], xb[:, hf:]])
        return seg(x3, src2d, dst2d)

    agg0 = segsum(ft_lv0)
    ft1 = _dense(_dense1_body, ft_lv0, agg0, W1, 2000)
    agg1 = segsum(ft1)
    out = _dense(_dense2_body, ft1, agg1, W2, 2000)
    return out
